# baseline (speedup 1.0000x reference)
import functools

import jax
import jax.numpy as jnp
from jax import lax
from jax.experimental import pallas as pl
from jax.experimental.pallas import tpu as pltpu
from jax.experimental.pallas import tpu_sc as plsc

N = 4096
M = 4096
E = 64
BB = 256
_C = M // 128
_XR = BB // 2

_SC_INFO = plsc.get_sparse_core_info()
_NC = _SC_INFO.num_cores
_NS = _SC_INFO.num_subcores
_NW = _NC * _NS


def _make_sc_gather(total):
    per_w = total // _NW
    mesh = plsc.VectorSubcoreMesh(core_axis_name="c", subcore_axis_name="s")

    @functools.partial(
        pl.kernel,
        mesh=mesh,
        out_type=jax.ShapeDtypeStruct((total,), jnp.float32),
        scratch_types=[
            pltpu.VMEM((per_w,), jnp.int32),
            pltpu.VMEM((per_w,), jnp.float32),
            pltpu.SemaphoreType.DMA,
        ],
    )
    def gather_k(xflat_hbm, idx_hbm, out_hbm, idx_v, val_v, sem):
        wid = lax.axis_index("s") * _NC + lax.axis_index("c")
        base = pl.multiple_of(wid * per_w, 8)
        pltpu.sync_copy(idx_hbm.at[pl.ds(base, per_w)], idx_v)
        pltpu.async_copy(xflat_hbm.at[idx_v], val_v, sem).wait()
        pltpu.sync_copy(val_v, out_hbm.at[pl.ds(base, per_w)])

    return gather_k


def _tile_body(dstdup_ref, dstcol_ref, valscol_ref, bias_ref, xg_ref, out_ref):
    dstdup = dstdup_ref[...]
    dstcol = dstcol_ref[...]
    valscol = valscol_ref[...]

    j_iota = jax.lax.broadcasted_iota(jnp.int32, (2 * E, 128), 1)
    s_mat = jnp.where(j_iota == dstcol % 128, valscol, 0.0)

    q_iota = jax.lax.broadcasted_iota(jnp.int32, (2 * _C, 2 * E), 0)
    f_iota = jax.lax.broadcasted_iota(jnp.int32, (2 * _C, 2 * E), 1)
    bigmask = (
        (f_iota // E == q_iota // _C) & (dstdup // 128 == q_iota % _C)
    ).astype(jnp.float32)

    xg2 = xg_ref[...]
    xg4 = (xg2[:, None, :] * bigmask[None, :, :]).reshape(_XR * 2 * _C, 2 * E)
    part = jax.lax.dot_general(
        xg4, s_mat,
        dimension_numbers=(((1,), (0,)), ((), ())),
        preferred_element_type=jnp.float32,
    )

    bias2 = jnp.concatenate([bias_ref[...], bias_ref[...]], axis=0)
    bias_blk = jnp.broadcast_to(bias2[None], (_XR, 2 * _C, 128))
    out_ref[...] = part + bias_blk.reshape(BB * _C, 128)


@jax.jit
def kernel(x, indices, values, bias):
    b = x.shape[0]
    xflat = x.reshape(b * N)
    nr = b * E // 128
    rr = jax.lax.broadcasted_iota(jnp.int32, (nr, 128), 0)
    half = jax.lax.broadcasted_iota(jnp.int32, (nr, 128), 1) // E
    src2 = jnp.tile(indices[0], 2)[None, :]
    flat_idx = ((2 * rr + half) * N + src2).reshape(b * E)
    xg2d = _make_sc_gather(b * E)(xflat, flat_idx).reshape(nr, 128)

    dst2 = jnp.tile(indices[1], 2)
    dstdup = dst2.reshape(1, 2 * E)
    dstcol = dst2.reshape(2 * E, 1)
    valscol = jnp.tile(values, 2).reshape(2 * E, 1)
    bias32 = bias.reshape(_C, 128)
    out = pl.pallas_call(
        _tile_body,
        grid=(b // BB,),
        in_specs=[
            pl.BlockSpec((1, 2 * E), lambda i: (0, 0)),
            pl.BlockSpec((2 * E, 1), lambda i: (0, 0)),
            pl.BlockSpec((2 * E, 1), lambda i: (0, 0)),
            pl.BlockSpec((_C, 128), lambda i: (0, 0)),
            pl.BlockSpec((_XR, 128), lambda i: (i, 0)),
        ],
        out_specs=pl.BlockSpec((BB * _C, 128), lambda i: (i, 0)),
        out_shape=jax.ShapeDtypeStruct((b * _C, 128), jnp.float32),
    )(dstdup, dstcol, valscol, bias32, xg2d)
    return out.reshape(b, M, 1)
